# baseline (speedup 1.0000x reference)
import functools

import jax
import jax.numpy as jnp
from jax import lax
from jax.experimental import pallas as pl
from jax.experimental.pallas import tpu as pltpu
from jax.experimental.pallas import tpu_sc as plsc

D = 128
N_ROWS = 100000
NC, NS = 2, 16
NW = NC * NS

CHUNK_ROWS = 125
CHUNK_F = CHUNK_ROWS * D
NLANE = 8
NBUF = 4
SC_CHUNKS = 8
SC_ROWS = NW * CHUNK_ROWS * SC_CHUNKS
NSUPER = SC_CHUNKS // NBUF

TC_OFF = 0
TC_ROWS = N_ROWS - TC_OFF
TC_BLOCK = 5000
TC_GRID = TC_ROWS // TC_BLOCK


def _sc_partial_sums(nbr_flat):
    mesh = plsc.VectorSubcoreMesh(
        core_axis_name="c", subcore_axis_name="s", num_cores=NC, num_subcores=NS
    )

    @functools.partial(
        pl.kernel,
        out_type=jax.ShapeDtypeStruct((NW, D), jnp.float32),
        mesh=mesh,
        scratch_types=[pltpu.VMEM((CHUNK_F,), jnp.float32)] * NBUF
        + [pltpu.VMEM((D,), jnp.float32)]
        + [pltpu.SemaphoreType.DMA] * NBUF,
    )
    def k(nbr_hbm, part_hbm, *scratch):
        bufs = scratch[:NBUF]
        accv = scratch[NBUF]
        sems = scratch[NBUF + 1 :]
        cid = lax.axis_index("c")
        sid = lax.axis_index("s")
        wid = sid * NC + cid
        base = wid * (SC_CHUNKS * CHUNK_F)

        def start(chunk, b):
            pltpu.async_copy(
                nbr_hbm.at[pl.ds(base + chunk * CHUNK_F, CHUNK_F)], bufs[b], sems[b]
            )

        def wait(b):
            pltpu.make_async_copy(
                nbr_hbm.at[pl.ds(base, CHUNK_F)], bufs[b], sems[b]
            ).wait()

        def accumulate(b, accs):
            def row_body(r, accs):
                off = r * D
                return tuple(
                    a + bufs[b][pl.ds(off + k * 16, 16)] for k, a in enumerate(accs)
                )

            return lax.fori_loop(0, CHUNK_ROWS, row_body, accs, unroll=5)

        for b in range(NBUF):
            start(b, b)

        def superchunk(si, accs, fire):
            for b in range(NBUF):
                wait(b)
                accs = accumulate(b, accs)
                if fire:
                    start(si * NBUF + b + NBUF, b)
            return accs

        accs = tuple(jnp.zeros((16,), jnp.float32) for _ in range(NLANE))
        accs = lax.fori_loop(
            0, NSUPER - 1, lambda si, a: superchunk(si, a, True), accs
        )
        accs = superchunk(NSUPER - 1, accs, False)
        for k_i in range(NLANE):
            accv[pl.ds(k_i * 16, 16)] = accs[k_i]
        pltpu.sync_copy(accv, part_hbm.at[wid])

    return k(nbr_flat)


def _tc_reduce_body(nbr_ref, out_ref, acc_ref):
    part = jnp.sum(nbr_ref[...].reshape(TC_BLOCK // 8, 8, D), axis=0)

    @pl.when(pl.program_id(0) == 0)
    def _init():
        acc_ref[...] = part

    @pl.when(pl.program_id(0) != 0)
    def _acc():
        acc_ref[...] += part

    @pl.when(pl.program_id(0) == TC_GRID - 1)
    def _fin():
        out_ref[...] = jnp.sum(acc_ref[...], axis=0, keepdims=True)


def _tc_reduce(nbr_full):
    blk_off = TC_OFF // TC_BLOCK
    return pl.pallas_call(
        _tc_reduce_body,
        grid=(TC_GRID,),
        in_specs=[pl.BlockSpec((TC_BLOCK, D), lambda i: (i + blk_off, 0))],
        out_specs=pl.BlockSpec((1, D), lambda i: (0, 0)),
        out_shape=jax.ShapeDtypeStruct((1, D), jnp.float32),
        scratch_shapes=[pltpu.VMEM((8, D), jnp.float32)],
    )(nbr_full)


def _tc_finish_body(
    part_sc_ref, part_tc_ref, node_ref, w0t_ref, b0_ref, w1t_ref, b1_ref, out_ref
):
    total = jnp.sum(part_sc_ref[...], axis=0, keepdims=True) + part_tc_ref[...]
    mean = total * (1.0 / N_ROWS)
    node = node_ref[...]
    h = (
        jnp.dot(node, w0t_ref[:D, :], preferred_element_type=jnp.float32)
        + jnp.dot(mean, w0t_ref[D:, :], preferred_element_type=jnp.float32)
        + b0_ref[...]
    )
    h = jnp.maximum(h, 0.0)
    out = jnp.dot(h, w1t_ref[...], preferred_element_type=jnp.float32) + b1_ref[...]
    out_ref[...] = jnp.maximum(out, 0.0)


def _tc_finish(part_sc, part_tc, node2, w0t, b02, w1t, b12):
    return pl.pallas_call(
        _tc_finish_body,
        out_shape=jax.ShapeDtypeStruct((1, D), jnp.float32),
    )(part_sc, part_tc, node2, w0t, b02, w1t, b12)


def kernel(node_features, neighbor_features_list, W0, b0, W1, b1):
    nbr = neighbor_features_list.reshape(N_ROWS, D)
    part_sc = jnp.zeros((NW, D), jnp.float32)
    part_tc = _tc_reduce(nbr)
    out = _tc_finish(
        part_sc,
        part_tc,
        node_features.reshape(1, D),
        W0.T,
        b0.reshape(1, -1),
        W1.T,
        b1.reshape(1, -1),
    )
    return out.reshape(D)


# speedup vs baseline: 1.4472x; 1.4472x over previous
import functools

import jax
import jax.numpy as jnp
from jax import lax
from jax.experimental import pallas as pl
from jax.experimental.pallas import tpu as pltpu
from jax.experimental.pallas import tpu_sc as plsc

D = 128
N_ROWS = 100000
NC, NS = 2, 16
NW = NC * NS

CHUNK_ROWS = 125
CHUNK_F = CHUNK_ROWS * D
NLANE = 8
NBUF = 4
SC_CHUNKS = 8
SC_ROWS = NW * CHUNK_ROWS * SC_CHUNKS
NSUPER = SC_CHUNKS // NBUF

TC_OFF = 0
TC_ROWS = N_ROWS - TC_OFF
TC_WAYS = 4
TC_BLOCK = 5000
TC_GRID = TC_ROWS // (TC_BLOCK * TC_WAYS)


def _sc_partial_sums(nbr_flat):
    mesh = plsc.VectorSubcoreMesh(
        core_axis_name="c", subcore_axis_name="s", num_cores=NC, num_subcores=NS
    )

    @functools.partial(
        pl.kernel,
        out_type=jax.ShapeDtypeStruct((NW, D), jnp.float32),
        mesh=mesh,
        scratch_types=[pltpu.VMEM((CHUNK_F,), jnp.float32)] * NBUF
        + [pltpu.VMEM((D,), jnp.float32)]
        + [pltpu.SemaphoreType.DMA] * NBUF,
    )
    def k(nbr_hbm, part_hbm, *scratch):
        bufs = scratch[:NBUF]
        accv = scratch[NBUF]
        sems = scratch[NBUF + 1 :]
        cid = lax.axis_index("c")
        sid = lax.axis_index("s")
        wid = sid * NC + cid
        base = wid * (SC_CHUNKS * CHUNK_F)

        def start(chunk, b):
            pltpu.async_copy(
                nbr_hbm.at[pl.ds(base + chunk * CHUNK_F, CHUNK_F)], bufs[b], sems[b]
            )

        def wait(b):
            pltpu.make_async_copy(
                nbr_hbm.at[pl.ds(base, CHUNK_F)], bufs[b], sems[b]
            ).wait()

        def accumulate(b, accs):
            def row_body(r, accs):
                off = r * D
                return tuple(
                    a + bufs[b][pl.ds(off + k * 16, 16)] for k, a in enumerate(accs)
                )

            return lax.fori_loop(0, CHUNK_ROWS, row_body, accs, unroll=5)

        for b in range(NBUF):
            start(b, b)

        def superchunk(si, accs, fire):
            for b in range(NBUF):
                wait(b)
                accs = accumulate(b, accs)
                if fire:
                    start(si * NBUF + b + NBUF, b)
            return accs

        accs = tuple(jnp.zeros((16,), jnp.float32) for _ in range(NLANE))
        accs = lax.fori_loop(
            0, NSUPER - 1, lambda si, a: superchunk(si, a, True), accs
        )
        accs = superchunk(NSUPER - 1, accs, False)
        for k_i in range(NLANE):
            accv[pl.ds(k_i * 16, 16)] = accs[k_i]
        pltpu.sync_copy(accv, part_hbm.at[wid])

    return k(nbr_flat)


def _tc_reduce_body(*refs):
    nbr_refs = refs[:TC_WAYS]
    out_ref = refs[TC_WAYS]
    acc_ref = refs[TC_WAYS + 1]
    part = sum(
        jnp.sum(r[...].reshape(TC_BLOCK // 8, 8, D), axis=0) for r in nbr_refs
    )

    @pl.when(pl.program_id(0) == 0)
    def _init():
        acc_ref[...] = part

    @pl.when(pl.program_id(0) != 0)
    def _acc():
        acc_ref[...] += part

    @pl.when(pl.program_id(0) == TC_GRID - 1)
    def _fin():
        out_ref[...] = jnp.sum(acc_ref[...], axis=0, keepdims=True)


def _tc_reduce(nbr_full):
    blk_off = TC_OFF // TC_BLOCK
    specs = [
        pl.BlockSpec(
            (TC_BLOCK, D),
            functools.partial(lambda j, i: (i + blk_off + j * TC_GRID, 0), j),
        )
        for j in range(TC_WAYS)
    ]
    return pl.pallas_call(
        _tc_reduce_body,
        grid=(TC_GRID,),
        in_specs=specs,
        out_specs=pl.BlockSpec((1, D), lambda i: (0, 0)),
        out_shape=jax.ShapeDtypeStruct((1, D), jnp.float32),
        scratch_shapes=[pltpu.VMEM((8, D), jnp.float32)],
    )(*([nbr_full] * TC_WAYS))


def _tc_finish_body(
    part_sc_ref, part_tc_ref, node_ref, w0t_ref, b0_ref, w1t_ref, b1_ref, out_ref
):
    total = jnp.sum(part_sc_ref[...], axis=0, keepdims=True) + part_tc_ref[...]
    mean = total * (1.0 / N_ROWS)
    node = node_ref[...]
    h = (
        jnp.dot(node, w0t_ref[:D, :], preferred_element_type=jnp.float32)
        + jnp.dot(mean, w0t_ref[D:, :], preferred_element_type=jnp.float32)
        + b0_ref[...]
    )
    h = jnp.maximum(h, 0.0)
    out = jnp.dot(h, w1t_ref[...], preferred_element_type=jnp.float32) + b1_ref[...]
    out_ref[...] = jnp.maximum(out, 0.0)


def _tc_finish(part_sc, part_tc, node2, w0t, b02, w1t, b12):
    return pl.pallas_call(
        _tc_finish_body,
        out_shape=jax.ShapeDtypeStruct((1, D), jnp.float32),
    )(part_sc, part_tc, node2, w0t, b02, w1t, b12)


def kernel(node_features, neighbor_features_list, W0, b0, W1, b1):
    nbr = neighbor_features_list.reshape(N_ROWS, D)
    part_sc = jnp.zeros((NW, D), jnp.float32)
    part_tc = _tc_reduce(nbr)
    out = _tc_finish(
        part_sc,
        part_tc,
        node_features.reshape(1, D),
        W0.T,
        b0.reshape(1, -1),
        W1.T,
        b1.reshape(1, -1),
    )
    return out.reshape(D)


# speedup vs baseline: 1.4691x; 1.0151x over previous
import functools

import jax
import jax.numpy as jnp
from jax import lax
from jax.experimental import pallas as pl
from jax.experimental.pallas import tpu as pltpu
from jax.experimental.pallas import tpu_sc as plsc

D = 128
N_ROWS = 100000
NC, NS = 2, 16
NW = NC * NS

CHUNK_ROWS = 125
CHUNK_F = CHUNK_ROWS * D
NLANE = 8
NBUF = 4
SC_CHUNKS = 8
SC_ROWS = NW * CHUNK_ROWS * SC_CHUNKS
NSUPER = SC_CHUNKS // NBUF

TC_OFF = 0
TC_ROWS = N_ROWS - TC_OFF
TC_WAYS = 10
TC_BLOCK = 2000
TC_GRID = TC_ROWS // (TC_BLOCK * TC_WAYS)


def _sc_partial_sums(nbr_flat):
    mesh = plsc.VectorSubcoreMesh(
        core_axis_name="c", subcore_axis_name="s", num_cores=NC, num_subcores=NS
    )

    @functools.partial(
        pl.kernel,
        out_type=jax.ShapeDtypeStruct((NW, D), jnp.float32),
        mesh=mesh,
        scratch_types=[pltpu.VMEM((CHUNK_F,), jnp.float32)] * NBUF
        + [pltpu.VMEM((D,), jnp.float32)]
        + [pltpu.SemaphoreType.DMA] * NBUF,
    )
    def k(nbr_hbm, part_hbm, *scratch):
        bufs = scratch[:NBUF]
        accv = scratch[NBUF]
        sems = scratch[NBUF + 1 :]
        cid = lax.axis_index("c")
        sid = lax.axis_index("s")
        wid = sid * NC + cid
        base = wid * (SC_CHUNKS * CHUNK_F)

        def start(chunk, b):
            pltpu.async_copy(
                nbr_hbm.at[pl.ds(base + chunk * CHUNK_F, CHUNK_F)], bufs[b], sems[b]
            )

        def wait(b):
            pltpu.make_async_copy(
                nbr_hbm.at[pl.ds(base, CHUNK_F)], bufs[b], sems[b]
            ).wait()

        def accumulate(b, accs):
            def row_body(r, accs):
                off = r * D
                return tuple(
                    a + bufs[b][pl.ds(off + k * 16, 16)] for k, a in enumerate(accs)
                )

            return lax.fori_loop(0, CHUNK_ROWS, row_body, accs, unroll=5)

        for b in range(NBUF):
            start(b, b)

        def superchunk(si, accs, fire):
            for b in range(NBUF):
                wait(b)
                accs = accumulate(b, accs)
                if fire:
                    start(si * NBUF + b + NBUF, b)
            return accs

        accs = tuple(jnp.zeros((16,), jnp.float32) for _ in range(NLANE))
        accs = lax.fori_loop(
            0, NSUPER - 1, lambda si, a: superchunk(si, a, True), accs
        )
        accs = superchunk(NSUPER - 1, accs, False)
        for k_i in range(NLANE):
            accv[pl.ds(k_i * 16, 16)] = accs[k_i]
        pltpu.sync_copy(accv, part_hbm.at[wid])

    return k(nbr_flat)


def _tc_reduce_body(*refs):
    nbr_refs = refs[:TC_WAYS]
    out_ref = refs[TC_WAYS]
    acc_ref = refs[TC_WAYS + 1]
    part = sum(
        jnp.sum(r[...].reshape(TC_BLOCK // 8, 8, D), axis=0) for r in nbr_refs
    )

    @pl.when(pl.program_id(0) == 0)
    def _init():
        acc_ref[...] = part

    @pl.when(pl.program_id(0) != 0)
    def _acc():
        acc_ref[...] += part

    @pl.when(pl.program_id(0) == TC_GRID - 1)
    def _fin():
        out_ref[...] = jnp.sum(acc_ref[...], axis=0, keepdims=True)


def _tc_reduce(nbr_full):
    blk_off = TC_OFF // TC_BLOCK
    specs = [
        pl.BlockSpec(
            (TC_BLOCK, D),
            functools.partial(lambda j, i: (i + blk_off + j * TC_GRID, 0), j),
        )
        for j in range(TC_WAYS)
    ]
    return pl.pallas_call(
        _tc_reduce_body,
        grid=(TC_GRID,),
        in_specs=specs,
        out_specs=pl.BlockSpec((1, D), lambda i: (0, 0)),
        out_shape=jax.ShapeDtypeStruct((1, D), jnp.float32),
        scratch_shapes=[pltpu.VMEM((8, D), jnp.float32)],
    )(*([nbr_full] * TC_WAYS))


def _tc_finish_body(
    part_sc_ref, part_tc_ref, node_ref, w0t_ref, b0_ref, w1t_ref, b1_ref, out_ref
):
    total = jnp.sum(part_sc_ref[...], axis=0, keepdims=True) + part_tc_ref[...]
    mean = total * (1.0 / N_ROWS)
    node = node_ref[...]
    h = (
        jnp.dot(node, w0t_ref[:D, :], preferred_element_type=jnp.float32)
        + jnp.dot(mean, w0t_ref[D:, :], preferred_element_type=jnp.float32)
        + b0_ref[...]
    )
    h = jnp.maximum(h, 0.0)
    out = jnp.dot(h, w1t_ref[...], preferred_element_type=jnp.float32) + b1_ref[...]
    out_ref[...] = jnp.maximum(out, 0.0)


def _tc_finish(part_sc, part_tc, node2, w0t, b02, w1t, b12):
    return pl.pallas_call(
        _tc_finish_body,
        out_shape=jax.ShapeDtypeStruct((1, D), jnp.float32),
    )(part_sc, part_tc, node2, w0t, b02, w1t, b12)


def kernel(node_features, neighbor_features_list, W0, b0, W1, b1):
    nbr = neighbor_features_list.reshape(N_ROWS, D)
    part_sc = jnp.zeros((NW, D), jnp.float32)
    part_tc = _tc_reduce(nbr)
    out = _tc_finish(
        part_sc,
        part_tc,
        node_features.reshape(1, D),
        W0.T,
        b0.reshape(1, -1),
        W1.T,
        b1.reshape(1, -1),
    )
    return out.reshape(D)
